# baseline (device time: 107155 ns/iter reference)
import jax
import jax.numpy as jnp
from jax import lax
from jax.experimental import pallas as pl
from jax.experimental.pallas import tpu as pltpu

N_DEV = 4
SQ = 256
D_MODEL = 1024
H_LOC = 8
H_GLB = 32
DH = 128
QBLK = 64
N_QB = SQ // QBLK
N_KB = 16
KEYS = N_KB * QBLK
SCALE = 0.08838834764831843
BF16 = jnp.bfloat16


def kernel(x, Wq, K_ext, V_ext, Wo):
    x2 = x.reshape(SQ, D_MODEL)
    Kr = K_ext.reshape(N_KB, N_QB, QBLK, H_GLB, DH)
    Vr = V_ext.reshape(N_KB, N_QB, QBLK, H_GLB, DH)

    def body(x_ref, wq_ref, k_hbm, v_hbm, wo_ref, out_ref,
             kbuf, vbuf, ctx_ref, comm_ref, copy_sems, send_sems, recv_sems):
        my = lax.axis_index("i")
        left = (my + N_DEV - 1) % N_DEV
        right = (my + 1) % N_DEV
        h0 = my * H_LOC

        barrier = pltpu.get_barrier_semaphore()
        for nbr in (left, right):
            pl.semaphore_signal(barrier, inc=1, device_id=(nbr,),
                                device_id_type=pl.DeviceIdType.MESH)
        pl.semaphore_wait(barrier, 2)

        q = jnp.dot(x_ref[...].astype(BF16), wq_ref[...].astype(BF16),
                    preferred_element_type=jnp.float32) * SCALE

        for qb in range(N_QB):
            kcopy = pltpu.make_async_copy(
                k_hbm.at[:, qb, :, pl.ds(h0, H_LOC), :], kbuf, copy_sems.at[0])
            vcopy = pltpu.make_async_copy(
                v_hbm.at[:, qb, :, pl.ds(h0, H_LOC), :], vbuf, copy_sems.at[1])
            kcopy.start()
            vcopy.start()
            kcopy.wait()
            vcopy.wait()
            for h in range(H_LOC):
                q_h = q[qb * QBLK:(qb + 1) * QBLK, h * DH:(h + 1) * DH].astype(BF16)
                k_h = kbuf[:, :, h, :].reshape(KEYS, DH).astype(BF16)
                v_h = vbuf[:, :, h, :].reshape(KEYS, DH).astype(BF16)
                s = lax.dot_general(q_h, k_h, (((1,), (1,)), ((), ())),
                                    preferred_element_type=jnp.float32)
                m_ = jnp.max(s, axis=1, keepdims=True)
                w = jnp.exp(s - m_)
                w = w / jnp.sum(w, axis=1, keepdims=True)
                ctx_h = jnp.dot(w.astype(BF16), v_h,
                                preferred_element_type=jnp.float32)
                ctx_ref[qb * QBLK:(qb + 1) * QBLK,
                        h * DH:(h + 1) * DH] = ctx_h.astype(BF16)

        out_ref[...] = jnp.dot(ctx_ref[...], wo_ref[...].astype(BF16),
                               preferred_element_type=jnp.float32)

        for hop in range(N_DEV - 1):
            src = out_ref if hop == 0 else comm_ref.at[hop - 1]
            rdma = pltpu.make_async_remote_copy(
                src_ref=src,
                dst_ref=comm_ref.at[hop],
                send_sem=send_sems.at[hop],
                recv_sem=recv_sems.at[hop],
                device_id=(right,),
                device_id_type=pl.DeviceIdType.MESH,
            )
            rdma.start()
            rdma.wait()
            out_ref[...] += comm_ref[hop]

    out = pl.pallas_call(
        body,
        out_shape=jax.ShapeDtypeStruct((SQ, D_MODEL), jnp.float32),
        in_specs=[
            pl.BlockSpec(memory_space=pltpu.MemorySpace.VMEM),
            pl.BlockSpec(memory_space=pltpu.MemorySpace.VMEM),
            pl.BlockSpec(memory_space=pltpu.MemorySpace.HBM),
            pl.BlockSpec(memory_space=pltpu.MemorySpace.HBM),
            pl.BlockSpec(memory_space=pltpu.MemorySpace.VMEM),
        ],
        out_specs=pl.BlockSpec(memory_space=pltpu.MemorySpace.VMEM),
        scratch_shapes=[
            pltpu.VMEM((N_KB, QBLK, H_LOC, DH), jnp.float32),
            pltpu.VMEM((N_KB, QBLK, H_LOC, DH), jnp.float32),
            pltpu.VMEM((SQ, D_MODEL), BF16),
            pltpu.VMEM((N_DEV - 1, SQ, D_MODEL), jnp.float32),
            pltpu.SemaphoreType.DMA((2,)),
            pltpu.SemaphoreType.DMA((N_DEV - 1,)),
            pltpu.SemaphoreType.DMA((N_DEV - 1,)),
        ],
        compiler_params=pltpu.CompilerParams(collective_id=0),
    )(x2, Wq, Kr, Vr, Wo)
    return out.reshape(1, SQ, D_MODEL)


# device time: 70398 ns/iter; 1.5221x vs baseline; 1.5221x over previous
import jax
import jax.numpy as jnp
from jax import lax
from jax.experimental import pallas as pl
from jax.experimental.pallas import tpu as pltpu

N_DEV = 4
SQ = 256
D_MODEL = 1024
H_LOC = 8
H_GLB = 32
DH = 128
QBLK = 64
N_QB = SQ // QBLK
N_KB = 16
KEYS = N_KB * QBLK
SCALE = 0.08838834764831843
BF16 = jnp.bfloat16


def kernel(x, Wq, K_ext, V_ext, Wo):
    x2 = x.reshape(SQ, D_MODEL)
    Kr = K_ext.reshape(N_KB, N_QB, QBLK, H_GLB, DH)
    Vr = V_ext.reshape(N_KB, N_QB, QBLK, H_GLB, DH)

    def body(x_ref, wq_ref, k_hbm, v_hbm, wo_ref, out_ref,
             kbuf, vbuf, ctx_ref, pbuf, sbuf, comm_a, comm_b,
             kv_sems, ar_send, ar_recv):
        my = lax.axis_index("i")
        left = (my + N_DEV - 1) % N_DEV
        right = (my + 1) % N_DEV
        h0 = my * H_LOC

        barrier = pltpu.get_barrier_semaphore()
        for nbr in (left, right):
            pl.semaphore_signal(barrier, inc=1, device_id=(nbr,),
                                device_id_type=pl.DeviceIdType.MESH)
        pl.semaphore_wait(barrier, 2)

        def start_kv(qb, slot):
            kc = pltpu.make_async_copy(
                k_hbm.at[:, qb, :, pl.ds(h0, H_LOC), :],
                kbuf.at[slot], kv_sems.at[slot, 0])
            vc = pltpu.make_async_copy(
                v_hbm.at[:, qb, :, pl.ds(h0, H_LOC), :],
                vbuf.at[slot], kv_sems.at[slot, 1])
            kc.start()
            vc.start()
            return kc, vc

        pending = [start_kv(0, 0), start_kv(1, 1)]

        q = jnp.dot(x_ref[...].astype(BF16), wq_ref[...].astype(BF16),
                    preferred_element_type=jnp.float32) * SCALE

        for qb in range(N_QB):
            slot = qb % 2
            kc, vc = pending[qb]
            kc.wait()
            vc.wait()
            for h in range(H_LOC):
                q_h = q[qb * QBLK:(qb + 1) * QBLK, h * DH:(h + 1) * DH].astype(BF16)
                k_h = kbuf[slot, :, :, h, :].reshape(KEYS, DH).astype(BF16)
                v_h = vbuf[slot, :, :, h, :].reshape(KEYS, DH).astype(BF16)
                s = lax.dot_general(q_h, k_h, (((1,), (1,)), ((), ())),
                                    preferred_element_type=jnp.float32)
                m_ = jnp.max(s, axis=1, keepdims=True)
                w = jnp.exp(s - m_)
                w = w / jnp.sum(w, axis=1, keepdims=True)
                ctx_h = jnp.dot(w.astype(BF16), v_h,
                                preferred_element_type=jnp.float32)
                ctx_ref[qb * QBLK:(qb + 1) * QBLK,
                        h * DH:(h + 1) * DH] = ctx_h.astype(BF16)
            if qb + 2 < N_QB:
                pending.append(start_kv(qb + 2, slot))

        pbuf[...] = jnp.dot(ctx_ref[...], wo_ref[...].astype(BF16),
                            preferred_element_type=jnp.float32).astype(BF16)

        rdma_a = pltpu.make_async_remote_copy(
            src_ref=pbuf, dst_ref=comm_a,
            send_sem=ar_send.at[0], recv_sem=ar_recv.at[0],
            device_id=(my ^ 1,), device_id_type=pl.DeviceIdType.MESH,
        )
        rdma_a.start()
        rdma_a.wait()
        sbuf[...] = (pbuf[...].astype(jnp.float32)
                     + comm_a[...].astype(jnp.float32)).astype(BF16)

        rdma_b = pltpu.make_async_remote_copy(
            src_ref=sbuf, dst_ref=comm_b,
            send_sem=ar_send.at[1], recv_sem=ar_recv.at[1],
            device_id=(3 - my,), device_id_type=pl.DeviceIdType.MESH,
        )
        rdma_b.start()
        rdma_b.wait()
        out_ref[...] = (sbuf[...].astype(jnp.float32)
                        + comm_b[...].astype(jnp.float32))

    out = pl.pallas_call(
        body,
        out_shape=jax.ShapeDtypeStruct((SQ, D_MODEL), jnp.float32),
        in_specs=[
            pl.BlockSpec(memory_space=pltpu.MemorySpace.VMEM),
            pl.BlockSpec(memory_space=pltpu.MemorySpace.VMEM),
            pl.BlockSpec(memory_space=pltpu.MemorySpace.HBM),
            pl.BlockSpec(memory_space=pltpu.MemorySpace.HBM),
            pl.BlockSpec(memory_space=pltpu.MemorySpace.VMEM),
        ],
        out_specs=pl.BlockSpec(memory_space=pltpu.MemorySpace.VMEM),
        scratch_shapes=[
            pltpu.VMEM((2, N_KB, QBLK, H_LOC, DH), jnp.float32),
            pltpu.VMEM((2, N_KB, QBLK, H_LOC, DH), jnp.float32),
            pltpu.VMEM((SQ, D_MODEL), BF16),
            pltpu.VMEM((SQ, D_MODEL), BF16),
            pltpu.VMEM((SQ, D_MODEL), BF16),
            pltpu.VMEM((SQ, D_MODEL), BF16),
            pltpu.VMEM((SQ, D_MODEL), BF16),
            pltpu.SemaphoreType.DMA((2, 2)),
            pltpu.SemaphoreType.DMA((2,)),
            pltpu.SemaphoreType.DMA((2,)),
        ],
        compiler_params=pltpu.CompilerParams(collective_id=0),
    )(x2, Wq, Kr, Vr, Wo)
    return out.reshape(1, SQ, D_MODEL)
